# baseline (speedup 1.0000x reference)
import functools

import jax
import jax.numpy as jnp
from jax import lax
from jax.experimental import pallas as pl
from jax.experimental.pallas import tpu as pltpu
from jax.experimental.pallas import tpu_sc as plsc

_B, _C, _H, _W = 64, 768, 14, 14
_S = _H * _W
_E = 64
_NC, _NS = 2, 16
_NW = _NC * _NS
_ROWS = _B // _NW
_CH = 49
_NCHUNK = _S // _CH


def _sc_pool_body(x_hbm, out_hbm, buf, acc):
    wid = lax.axis_index("s") * _NC + lax.axis_index("c")
    r0 = _ROWS * wid

    for r in range(_ROWS):
        for l0 in range(0, _C, 16):
            acc[r, pl.ds(l0, 16)] = jnp.zeros((16,), jnp.float32)

    for chunk in range(_NCHUNK):
        pltpu.sync_copy(
            x_hbm.at[pl.ds(chunk * _CH, _CH), pl.ds(r0, _ROWS), :], buf
        )

        def _plane_add(p, carry):
            for r in range(_ROWS):
                for l0 in range(0, _C, 16):
                    plsc.addupdate(
                        acc.at[r, pl.ds(l0, 16)], buf[p, r, pl.ds(l0, 16)]
                    )
            return carry

        lax.fori_loop(0, _CH, _plane_add, 0)

    pltpu.sync_copy(acc, out_hbm.at[pl.ds(r0, _ROWS), :])


def _sc_pool(xp):
    return pl.kernel(
        _sc_pool_body,
        mesh=plsc.VectorSubcoreMesh(core_axis_name="c", subcore_axis_name="s"),
        out_type=jax.ShapeDtypeStruct((_B, _C), jnp.float32),
        scratch_types=[
            pltpu.VMEM((_CH, _ROWS, _C), jnp.float32),
            pltpu.VMEM((_ROWS, _C), jnp.float32),
        ],
    )(xp)


def _finish_body(acc_ref, w_ref, b_ref, o_ref):
    pooled = acc_ref[...] * (1.0 / _S)
    logits = lax.dot_general(
        pooled, w_ref[...], (((1,), (1,)), ((), ())),
        preferred_element_type=jnp.float32,
    ) + b_ref[...]
    mx = jnp.max(logits, axis=1, keepdims=True)
    e = jnp.exp(logits - mx)
    o_ref[...] = e / jnp.sum(e, axis=1, keepdims=True)


def kernel(x, W, b):
    xp = jnp.transpose(x, (2, 3, 0, 1)).reshape(_S, _B, _C)
    psum = _sc_pool(xp)
    b2 = b.reshape(1, _E)
    return pl.pallas_call(
        _finish_body,
        grid=(1,),
        in_specs=[
            pl.BlockSpec((_B, _C), lambda i: (0, 0)),
            pl.BlockSpec((_E, _C), lambda i: (0, 0)),
            pl.BlockSpec((1, _E), lambda i: (0, 0)),
        ],
        out_specs=pl.BlockSpec((_B, _E), lambda i: (0, 0)),
        out_shape=jax.ShapeDtypeStruct((_B, _E), jnp.float32),
    )(psum, W, b2)
